# baseline (speedup 1.0000x reference)
import functools

import jax
import jax.numpy as jnp
from jax import lax
from jax.experimental import pallas as pl
from jax.experimental.pallas import tpu as pltpu
from jax.experimental.pallas import tpu_sc as plsc


def _sc_geometry():
    try:
        info = plsc.get_sparse_core_info()
        return info.num_cores, info.num_subcores
    except Exception:
        return 2, 16


@functools.partial(jax.jit, static_argnames=("ns_story", "ns_query", "W", "E", "cs"))
def _bag_sum(story_idx, query_idx, table, pos, ns_story, ns_query, W, E, cs):
    NC, NS = _sc_geometry()
    nw = NC * NS
    nchunk_s = ns_story // (nw * cs)
    nchunk_q = ns_query // (nw * cs)
    mesh = plsc.VectorSubcoreMesh(core_axis_name="c", subcore_axis_name="s")

    @functools.partial(
        pl.kernel,
        mesh=mesh,
        compiler_params=pltpu.CompilerParams(use_tc_tiling_on_sc=False),
        out_type=(
            jax.ShapeDtypeStruct((ns_story * E,), jnp.float32),
            jax.ShapeDtypeStruct((ns_query * E,), jnp.float32),
        ),
        scratch_types=[
            pltpu.VMEM((2, cs * W), jnp.int32),
            pltpu.VMEM((2, cs * W, E), jnp.float32),
            pltpu.VMEM((32, E), jnp.float32),
            pltpu.VMEM((2, cs * E), jnp.float32),
            pltpu.SemaphoreType.DMA,
            pltpu.SemaphoreType.DMA,
            pltpu.SemaphoreType.DMA,
            pltpu.SemaphoreType.DMA,
            pltpu.SemaphoreType.DMA,
            pltpu.SemaphoreType.DMA,
        ],
    )
    def body(sidx_hbm, qidx_hbm, table_hbm, pos_hbm, out_s_hbm, out_q_hbm,
             idx_v, rows_v, pos_v, out_v,
             sem_i0, sem_i1, sem_g0, sem_g1, sem_o0, sem_o1):
        sem_i = (sem_i0, sem_i1)
        sem_g = (sem_g0, sem_g1)
        sem_o = (sem_o0, sem_o1)
        wid = lax.axis_index("s") * NC + lax.axis_index("c")
        pltpu.sync_copy(pos_hbm, pos_v)

        G = 2

        def compute(b):
            rows = rows_v.at[b]

            def seg_body(g, carry):
                s0 = g * G
                accs = [[jnp.zeros((16,), jnp.float32) for _ in range(4)]
                        for _ in range(G)]
                for w in range(W):
                    pv = [pos_v[w, pl.ds(a * 16, 16)] for a in range(4)]
                    for s in range(G):
                        row = (s0 + s) * W + w
                        for a in range(4):
                            x = rows[row, pl.ds(a * 16, 16)]
                            accs[s][a] = accs[s][a] + x * pv[a]
                for s in range(G):
                    sE = (s0 + s) * E
                    for a in range(4):
                        out_v[b, pl.ds(sE + a * 16, 16)] = accs[s][a]
                return carry

            lax.fori_loop(0, cs // G, seg_body, 0, unroll=2)

        def run_pipeline(idx_hbm, out_hbm, nchunk):
            seg0 = wid * (nchunk * cs)

            def start_idx(i, b):
                pltpu.make_async_copy(
                    idx_hbm.at[pl.ds((seg0 + i * cs) * W, cs * W)],
                    idx_v.at[b], sem_i[b]).start()

            def wait_idx(i, b):
                pltpu.make_async_copy(
                    idx_hbm.at[pl.ds((seg0 + i * cs) * W, cs * W)],
                    idx_v.at[b], sem_i[b]).wait()

            def start_gather(b):
                pltpu.make_async_copy(
                    table_hbm.at[idx_v.at[b]], rows_v.at[b], sem_g[b]).start()

            def wait_gather(b):
                pltpu.make_async_copy(
                    table_hbm.at[idx_v.at[b]], rows_v.at[b], sem_g[b]).wait()

            def out_copy(i, b):
                return pltpu.make_async_copy(
                    out_v.at[b], out_hbm.at[pl.ds((seg0 + i * cs) * E, cs * E)],
                    sem_o[b])

            start_idx(0, 0)
            wait_idx(0, 0)
            start_gather(0)

            @pl.when(nchunk > 1)
            def _():
                start_idx(1, 1)

            def pair_body(ci, carry):
                for b in range(2):
                    i = 2 * ci + b
                    wait_gather(b)

                    @pl.when(i + 2 < nchunk)
                    def _():
                        start_idx(i + 2, b)

                    @pl.when(i + 1 < nchunk)
                    def _():
                        wait_idx(i + 1, 1 - b)
                        start_gather(1 - b)

                    @pl.when(i >= 2)
                    def _():
                        out_copy(i - 2, b).wait()

                    compute(b)
                    out_copy(i, b).start()
                return carry

            lax.fori_loop(0, nchunk // 2, pair_body, 0)
            out_copy(nchunk - 2, 0).wait()
            out_copy(nchunk - 1, 1).wait()

        run_pipeline(sidx_hbm, out_s_hbm, nchunk_s)
        run_pipeline(qidx_hbm, out_q_hbm, nchunk_q)

    return body(story_idx, query_idx, table, pos)


def kernel(story, query, word_embed, pos_embed):
    B, S, W = story.shape
    V, E = word_embed.shape
    out_s, out_q = _bag_sum(
        story.reshape(-1), query.reshape(-1), word_embed, pos_embed,
        ns_story=B * S, ns_query=B, W=W, E=E, cs=32)
    return (out_s.reshape(B, S, E), out_q.reshape(B, E))


# speedup vs baseline: 1.0566x; 1.0566x over previous
import functools

import jax
import jax.numpy as jnp
from jax import lax
from jax.experimental import pallas as pl
from jax.experimental.pallas import tpu as pltpu
from jax.experimental.pallas import tpu_sc as plsc


def _sc_geometry():
    try:
        info = plsc.get_sparse_core_info()
        return info.num_cores, info.num_subcores
    except Exception:
        return 2, 16


@functools.partial(jax.jit, static_argnames=("ns_story", "ns_query", "W", "E", "cs"))
def _bag_sum(story_idx, query_idx, table, pos, ns_story, ns_query, W, E, cs):
    NC, NS = _sc_geometry()
    nw = NC * NS
    nchunk_s = ns_story // (nw * cs)
    nchunk_q = ns_query // (nw * cs)
    mesh = plsc.VectorSubcoreMesh(core_axis_name="c", subcore_axis_name="s")

    @functools.partial(
        pl.kernel,
        mesh=mesh,
        compiler_params=pltpu.CompilerParams(use_tc_tiling_on_sc=False),
        out_type=(
            jax.ShapeDtypeStruct((ns_story * E,), jnp.float32),
            jax.ShapeDtypeStruct((ns_query * E,), jnp.float32),
        ),
        scratch_types=[
            pltpu.VMEM((2, cs * W), jnp.int32),
            pltpu.VMEM((2, cs * W, E), jnp.float32),
            pltpu.VMEM((32, E), jnp.float32),
            pltpu.VMEM((2, cs * E), jnp.float32),
            pltpu.SemaphoreType.DMA,
            pltpu.SemaphoreType.DMA,
            pltpu.SemaphoreType.DMA,
            pltpu.SemaphoreType.DMA,
            pltpu.SemaphoreType.DMA,
            pltpu.SemaphoreType.DMA,
        ],
    )
    def body(sidx_hbm, qidx_hbm, table_hbm, pos_hbm, out_s_hbm, out_q_hbm,
             idx_v, rows_v, pos_v, out_v,
             sem_i0, sem_i1, sem_g0, sem_g1, sem_o0, sem_o1):
        sem_i = (sem_i0, sem_i1)
        sem_g = (sem_g0, sem_g1)
        sem_o = (sem_o0, sem_o1)
        wid = lax.axis_index("s") * NC + lax.axis_index("c")
        pltpu.sync_copy(pos_hbm, pos_v)

        def compute(b):
            rows = rows_v.at[b]
            pvs = [[pos_v[w, pl.ds(a * 16, 16)] for w in range(W)] for a in range(4)]

            def seg_body(s, carry):
                base = s * W
                accs = [jnp.zeros((16,), jnp.float32) for _ in range(4)]
                acc2 = [jnp.zeros((16,), jnp.float32) for _ in range(4)]
                for w in range(W):
                    tgt = accs if (w % 2 == 0) else acc2
                    for a in range(4):
                        x = rows[base + w, pl.ds(a * 16, 16)]
                        tgt[a] = tgt[a] + x * pvs[a][w]
                sE = s * E
                for a in range(4):
                    out_v[b, pl.ds(sE + a * 16, 16)] = accs[a] + acc2[a]
                return carry

            lax.fori_loop(0, cs, seg_body, 0, unroll=4)

        def run_pipeline(idx_hbm, out_hbm, nchunk):
            seg0 = wid * (nchunk * cs)

            def start_idx(i, b):
                pltpu.make_async_copy(
                    idx_hbm.at[pl.ds((seg0 + i * cs) * W, cs * W)],
                    idx_v.at[b], sem_i[b]).start()

            def wait_idx(i, b):
                pltpu.make_async_copy(
                    idx_hbm.at[pl.ds((seg0 + i * cs) * W, cs * W)],
                    idx_v.at[b], sem_i[b]).wait()

            def start_gather(b):
                pltpu.make_async_copy(
                    table_hbm.at[idx_v.at[b]], rows_v.at[b], sem_g[b]).start()

            def wait_gather(b):
                pltpu.make_async_copy(
                    table_hbm.at[idx_v.at[b]], rows_v.at[b], sem_g[b]).wait()

            def out_copy(i, b):
                return pltpu.make_async_copy(
                    out_v.at[b], out_hbm.at[pl.ds((seg0 + i * cs) * E, cs * E)],
                    sem_o[b])

            start_idx(0, 0)
            wait_idx(0, 0)
            start_gather(0)

            @pl.when(nchunk > 1)
            def _():
                start_idx(1, 1)

            def pair_body(ci, carry):
                for b in range(2):
                    i = 2 * ci + b
                    wait_gather(b)

                    @pl.when(i + 2 < nchunk)
                    def _():
                        start_idx(i + 2, b)

                    @pl.when(i + 1 < nchunk)
                    def _():
                        wait_idx(i + 1, 1 - b)
                        start_gather(1 - b)

                    @pl.when(i >= 2)
                    def _():
                        out_copy(i - 2, b).wait()

                    compute(b)
                    out_copy(i, b).start()
                return carry

            lax.fori_loop(0, nchunk // 2, pair_body, 0)
            out_copy(nchunk - 2, 0).wait()
            out_copy(nchunk - 1, 1).wait()

        run_pipeline(sidx_hbm, out_s_hbm, nchunk_s)
        run_pipeline(qidx_hbm, out_q_hbm, nchunk_q)

    return body(story_idx, query_idx, table, pos)


def kernel(story, query, word_embed, pos_embed):
    B, S, W = story.shape
    V, E = word_embed.shape
    out_s, out_q = _bag_sum(
        story.reshape(-1), query.reshape(-1), word_embed, pos_embed,
        ns_story=B * S, ns_query=B, W=W, E=E, cs=32)
    return (out_s.reshape(B, S, E), out_q.reshape(B, E))
